# baseline (speedup 1.0000x reference)
import functools

import jax
import jax.numpy as jnp
from jax import lax
from jax.experimental import pallas as pl
from jax.experimental.pallas import tpu as pltpu
from jax.experimental.pallas import tpu_sc as plsc

_ALPHA = 0.1
_ITERS = 4
_B = 8
_N = 4096
_M = 2048
_L = 16
_NSUB = 16
_NCORE = 2
_QPER = _N * _B // (_NSUB * _NCORE)
_GRP = 4
_NBLK = _QPER // (_L * _GRP)

_GDN = jax.lax.GatherDimensionNumbers(
    offset_dims=(), collapsed_slice_dims=(0,), start_index_map=(0,))


def _lane_bcast(v, j):
    idx = jnp.full((_L,), j, dtype=jnp.int32)
    return lax.gather(v, idx[:, None], dimension_numbers=_GDN,
                      slice_sizes=(1,),
                      mode=lax.GatherScatterMode.PROMISE_IN_BOUNDS)


def _bf(x):
    i = lax.bitcast_convert_type(x, jnp.int32)
    i = i + jnp.int32(0x7FFF) + ((i >> 16) & 1)
    i = i & jnp.int32(-0x10000)
    return lax.bitcast_convert_type(i, jnp.float32)


def _sqrt16(x):
    i = lax.bitcast_convert_type(x, jnp.int32)
    i = jnp.int32(0x5F3759DF) - (i >> 1)
    r = lax.bitcast_convert_type(i, jnp.float32)
    for _ in range(3):
        r = r * (1.5 - 0.5 * x * r * r)
    return x * r


def _sc_body(pred_ref, part_ref, out_ref,
             qbuf, kbuf, kx, ky, kz, kxb, kyb, kzb, b2,
             qx, qy, qz, nx, ny, nz, nd, stage, grp, shared):
    c = lax.axis_index("c")
    s = lax.axis_index("s")
    b = c * 4 + s // 4
    qpart = s % 4
    qoff = qpart * (_QPER * 3)

    iota = lax.iota(jnp.int32, _L)
    iota3 = iota * 3

    pltpu.sync_copy(part_ref.at[b], kbuf)

    def key_prep(i, _):
        base = jnp.full((_L,), i * 48, dtype=jnp.int32) + iota3
        vx = plsc.load_gather(kbuf, [base])
        vy = plsc.load_gather(kbuf, [base + 1])
        vz = plsc.load_gather(kbuf, [base + 2])
        sl = pl.ds(i * _L, _L)
        kx[sl] = vx
        ky[sl] = vy
        kz[sl] = vz
        kxb[sl] = _bf(vx)
        kyb[sl] = _bf(vy)
        kzb[sl] = _bf(vz)
        b2[sl] = (vx * vx + vy * vy) + vz * vz
        return 0

    lax.fori_loop(0, _M // _L, key_prep, 0)

    pltpu.sync_copy(pred_ref.at[b, pl.ds(qoff, _QPER * 3)], qbuf)

    def q_prep(i, _):
        base = jnp.full((_L,), i * 48, dtype=jnp.int32) + iota3
        sl = pl.ds(i * _L, _L)
        qx[sl] = plsc.load_gather(qbuf, [base])
        qy[sl] = plsc.load_gather(qbuf, [base + 1])
        qz[sl] = plsc.load_gather(qbuf, [base + 2])
        return 0

    lax.fori_loop(0, _QPER // _L, q_prep, 0)

    for it in range(_ITERS):
        def scan_block(blk, mdv):
            q0 = blk * (_GRP * _L)
            sls = [pl.ds(q0 + g * _L, _L) for g in range(_GRP)]
            qxg = [qx[sl] for sl in sls]
            qyg = [qy[sl] for sl in sls]
            qzg = [qz[sl] for sl in sls]
            a2 = [(x * x + y * y) + z * z
                  for x, y, z in zip(qxg, qyg, qzg)]
            qxr = [_bf(x) for x in qxg]
            qyr = [_bf(y) for y in qyg]
            qzr = [_bf(z) for z in qzg]

            def chunk(ch, carry):
                bt, bi = carry
                bt = list(bt)
                bi = list(bi)
                ksl = pl.ds(ch * _L, _L)
                kvx = kxb[ksl]
                kvy = kyb[ksl]
                kvz = kzb[ksl]
                kv2 = b2[ksl]
                basev = jnp.full((_L,), ch * _L, dtype=jnp.int32)
                for j in range(_L):
                    xj = _lane_bcast(kvx, j)
                    yj = _lane_bcast(kvy, j)
                    zj = _lane_bcast(kvz, j)
                    b2j = _lane_bcast(kv2, j)
                    kid = basev + j
                    for g in range(_GRP):
                        dot = qxr[g] * xj + qyr[g] * yj + qzr[g] * zj
                        d2 = (a2[g] + b2j) - 2.0 * dot
                        m = d2 < bt[g]
                        bt[g] = jnp.where(m, d2, bt[g])
                        bi[g] = jnp.where(m, kid, bi[g])
                return tuple(bt), tuple(bi)


            bt0 = tuple(jnp.full((_L,), jnp.inf, jnp.float32)
                        for _ in range(_GRP))
            bi0 = tuple(jnp.zeros((_L,), jnp.int32) for _ in range(_GRP))
            bt, bi = lax.fori_loop(0, _M // _L, chunk, (bt0, bi0))

            for g in range(_GRP):
                d = _sqrt16(jnp.maximum(bt[g], 1e-12))
                nx[sls[g]] = plsc.load_gather(kx, [bi[g]])
                ny[sls[g]] = plsc.load_gather(ky, [bi[g]])
                nz[sls[g]] = plsc.load_gather(kz, [bi[g]])
                nd[sls[g]] = d
                mdv = jnp.maximum(mdv, d)
            return mdv

        mdv = lax.fori_loop(0, _NBLK, scan_block,
                            jnp.zeros((_L,), jnp.float32))

        stage[...] = mdv
        pltpu.sync_copy(stage, shared.at[s])
        plsc.subcore_barrier()
        gb = (s // 4) * 4
        pltpu.sync_copy(shared.at[pl.ds(gb, 4)], grp)
        m01 = jnp.maximum(grp[0], grp[1])
        m23 = jnp.maximum(grp[2], grp[3])
        mall = jnp.maximum(m01, m23)
        cm = plsc.cummax(mall)
        dmax = _lane_bcast(cm, _L - 1)
        plsc.subcore_barrier()
        denom = dmax + 1e-6

        def blend(u, _):
            sl = pl.ds(u * _L, _L)
            alpha = _ALPHA * (2.0 - nd[sl] / denom)
            qx[sl] = qx[sl] + alpha * (nx[sl] - qx[sl])
            qy[sl] = qy[sl] + alpha * (ny[sl] - qy[sl])
            qz[sl] = qz[sl] + alpha * (nz[sl] - qz[sl])
            return 0

        lax.fori_loop(0, _QPER // _L, blend, 0)

    def out_prep(u, _):
        base = jnp.full((_L,), u * 48, dtype=jnp.int32) + iota3
        sl = pl.ds(u * _L, _L)
        plsc.store_scatter(qbuf, [base], qx[sl])
        plsc.store_scatter(qbuf, [base + 1], qy[sl])
        plsc.store_scatter(qbuf, [base + 2], qz[sl])
        return 0

    lax.fori_loop(0, _QPER // _L, out_prep, 0)
    pltpu.sync_copy(qbuf, out_ref.at[b, pl.ds(qoff, _QPER * 3)])


@jax.jit
def kernel(pred, partial):
    pred2 = pred.reshape(_B, _N * 3)
    part2 = partial.reshape(_B, _M * 3)
    mesh = plsc.VectorSubcoreMesh(core_axis_name="c", subcore_axis_name="s")
    f = pl.kernel(
        _sc_body,
        out_type=jax.ShapeDtypeStruct((_B, _N * 3), jnp.float32),
        mesh=mesh,
        compiler_params=pltpu.CompilerParams(needs_layout_passes=False),
        scratch_types=[
            pltpu.VMEM((_QPER * 3,), jnp.float32),
            pltpu.VMEM((_M * 3,), jnp.float32),
            pltpu.VMEM((_M,), jnp.float32),
            pltpu.VMEM((_M,), jnp.float32),
            pltpu.VMEM((_M,), jnp.float32),
            pltpu.VMEM((_M,), jnp.float32),
            pltpu.VMEM((_M,), jnp.float32),
            pltpu.VMEM((_M,), jnp.float32),
            pltpu.VMEM((_M,), jnp.float32),
            pltpu.VMEM((_QPER,), jnp.float32),
            pltpu.VMEM((_QPER,), jnp.float32),
            pltpu.VMEM((_QPER,), jnp.float32),
            pltpu.VMEM((_QPER,), jnp.float32),
            pltpu.VMEM((_QPER,), jnp.float32),
            pltpu.VMEM((_QPER,), jnp.float32),
            pltpu.VMEM((_QPER,), jnp.float32),
            pltpu.VMEM((_L,), jnp.float32),
            pltpu.VMEM((4, _L), jnp.float32),
            pltpu.VMEM_SHARED((_NSUB, _L), jnp.float32),
        ],
    )
    out = f(pred2, part2)
    return out.reshape(_B, _N, 3)


# speedup vs baseline: 5.8819x; 5.8819x over previous
import functools

import jax
import jax.numpy as jnp
from jax import lax
from jax.experimental import pallas as pl
from jax.experimental.pallas import tpu as pltpu
from jax.experimental.pallas import tpu_sc as plsc

_ALPHA = 0.1
_ITERS = 4
_B = 8
_N = 4096
_M = 2048
_L = 16
_NSUB = 16
_NCORE = 2
_QPER = _N * _B // (_NSUB * _NCORE)
_NTILE = _QPER // _L

_GDN = jax.lax.GatherDimensionNumbers(
    offset_dims=(), collapsed_slice_dims=(0,), start_index_map=(0,))


def _lane_bcast(v, j):
    idx = jnp.full((_L,), j, dtype=jnp.int32)
    return lax.gather(v, idx[:, None], dimension_numbers=_GDN,
                      slice_sizes=(1,),
                      mode=lax.GatherScatterMode.PROMISE_IN_BOUNDS)


def _perm(v, idx):
    return lax.gather(v, idx[:, None], dimension_numbers=_GDN,
                      slice_sizes=(1,),
                      mode=lax.GatherScatterMode.PROMISE_IN_BOUNDS)


def _bf(x):
    i = lax.bitcast_convert_type(x, jnp.int32)
    i = i + jnp.int32(0x7FFF) + ((i >> 16) & 1)
    i = i & jnp.int32(-0x10000)
    return lax.bitcast_convert_type(i, jnp.float32)


def _sqrt16(x):
    i = lax.bitcast_convert_type(x, jnp.int32)
    i = jnp.int32(0x5F3759DF) - (i >> 1)
    r = lax.bitcast_convert_type(i, jnp.float32)
    for _ in range(3):
        r = r * (1.5 - 0.5 * x * r * r)
    return x * r


def _tree_min(v, perms):
    for p in perms:
        v = jnp.minimum(v, _perm(v, p))
    return v


def _sc_body(pred_ref, part_ref, out_ref,
             qbuf, kbuf, kx, ky, kz, kxm, kym, kzm, b2,
             qx, qy, qz, qxr, qyr, qzr, a2s,
             nx, ny, nz, nd, bix, stage, grp, shared):
    c = lax.axis_index("c")
    s = lax.axis_index("s")
    b = c * 4 + s // 4
    qpart = s % 4
    qoff = qpart * (_QPER * 3)

    iota = lax.iota(jnp.int32, _L)
    iota3 = iota * 3
    perms = [iota ^ (1 << k) for k in range(4)]
    lanesel = [iota == j for j in range(_L)]

    pltpu.sync_copy(part_ref.at[b], kbuf)

    def key_prep(i, _):
        base = jnp.full((_L,), i * 48, dtype=jnp.int32) + iota3
        vx = plsc.load_gather(kbuf, [base])
        vy = plsc.load_gather(kbuf, [base + 1])
        vz = plsc.load_gather(kbuf, [base + 2])
        sl = pl.ds(i * _L, _L)
        kx[sl] = vx
        ky[sl] = vy
        kz[sl] = vz
        kxm[sl] = -2.0 * _bf(vx)
        kym[sl] = -2.0 * _bf(vy)
        kzm[sl] = -2.0 * _bf(vz)
        b2[sl] = (vx * vx + vy * vy) + vz * vz
        return 0

    lax.fori_loop(0, _M // _L, key_prep, 0)

    pltpu.sync_copy(pred_ref.at[b, pl.ds(qoff, _QPER * 3)], qbuf)

    def q_prep(i, _):
        base = jnp.full((_L,), i * 48, dtype=jnp.int32) + iota3
        sl = pl.ds(i * _L, _L)
        vx = plsc.load_gather(qbuf, [base])
        vy = plsc.load_gather(qbuf, [base + 1])
        vz = plsc.load_gather(qbuf, [base + 2])
        qx[sl] = vx
        qy[sl] = vy
        qz[sl] = vz
        qxr[sl] = _bf(vx)
        qyr[sl] = _bf(vy)
        qzr[sl] = _bf(vz)
        a2s[sl] = (vx * vx + vy * vy) + vz * vz
        return 0

    lax.fori_loop(0, _QPER // _L, q_prep, 0)

    for it in range(_ITERS):
        def scan_tile(qt, _):
            sl = pl.ds(qt * _L, _L)
            qxv = qxr[sl]
            qyv = qyr[sl]
            qzv = qzr[sl]
            a2v = a2s[sl]
            res_d2 = jnp.zeros((_L,), jnp.float32)
            res_bi = jnp.zeros((_L,), jnp.int32)
            for pair in range(_L // 2):
                j0, j1 = 2 * pair, 2 * pair + 1
                x0 = _lane_bcast(qxv, j0)
                y0 = _lane_bcast(qyv, j0)
                z0 = _lane_bcast(qzv, j0)
                w0 = _lane_bcast(a2v, j0)
                x1 = _lane_bcast(qxv, j1)
                y1 = _lane_bcast(qyv, j1)
                z1 = _lane_bcast(qzv, j1)
                w1 = _lane_bcast(a2v, j1)

                def chunk(ch, carry):
                    bt0, bi0, bt1, bi1, idxv = carry
                    ksl = pl.ds(ch * _L, _L)
                    kvx = kxm[ksl]
                    kvy = kym[ksl]
                    kvz = kzm[ksl]
                    kv2 = b2[ksl]
                    d0 = (w0 + kv2) + ((x0 * kvx + y0 * kvy) + z0 * kvz)
                    m0 = d0 < bt0
                    bt0 = jnp.where(m0, d0, bt0)
                    bi0 = jnp.where(m0, idxv, bi0)
                    d1 = (w1 + kv2) + ((x1 * kvx + y1 * kvy) + z1 * kvz)
                    m1 = d1 < bt1
                    bt1 = jnp.where(m1, d1, bt1)
                    bi1 = jnp.where(m1, idxv, bi1)
                    return bt0, bi0, bt1, bi1, idxv + _L

                init = (jnp.full((_L,), jnp.inf, jnp.float32), iota,
                        jnp.full((_L,), jnp.inf, jnp.float32), iota, iota)
                bt0, bi0, bt1, bi1, _u = plsc.parallel_loop(
                    0, _M // _L, carry=init)(chunk)

                for jq, bt, bi in ((j0, bt0, bi0), (j1, bt1, bi1)):
                    mn = _tree_min(bt, perms)
                    cand = jnp.where(bt == mn, bi, jnp.int32(_M))
                    win = _tree_min(cand, perms)
                    res_d2 = jnp.where(lanesel[jq], mn, res_d2)
                    res_bi = jnp.where(lanesel[jq], win, res_bi)
            nd[sl] = res_d2
            bix[sl] = res_bi
            return 0

        lax.fori_loop(0, _NTILE, scan_tile, 0)

        def finish_tile(qt, mdv):
            sl = pl.ds(qt * _L, _L)
            biv = bix[sl]
            d = _sqrt16(jnp.maximum(nd[sl], 1e-12))
            nx[sl] = plsc.load_gather(kx, [biv])
            ny[sl] = plsc.load_gather(ky, [biv])
            nz[sl] = plsc.load_gather(kz, [biv])
            nd[sl] = d
            return jnp.maximum(mdv, d)

        mdv = lax.fori_loop(0, _NTILE, finish_tile,
                            jnp.zeros((_L,), jnp.float32))

        stage[...] = mdv
        pltpu.sync_copy(stage, shared.at[s])
        plsc.subcore_barrier()
        gb = (s // 4) * 4
        pltpu.sync_copy(shared.at[pl.ds(gb, 4)], grp)
        m01 = jnp.maximum(grp[0], grp[1])
        m23 = jnp.maximum(grp[2], grp[3])
        mall = _tree_min(-jnp.maximum(m01, m23), perms)
        dmax = -mall
        plsc.subcore_barrier()
        denom = dmax + 1e-6

        def blend(u, _):
            sl = pl.ds(u * _L, _L)
            alpha = _ALPHA * (2.0 - nd[sl] / denom)
            vx = qx[sl]
            vy = qy[sl]
            vz = qz[sl]
            vx = vx + alpha * (nx[sl] - vx)
            vy = vy + alpha * (ny[sl] - vy)
            vz = vz + alpha * (nz[sl] - vz)
            qx[sl] = vx
            qy[sl] = vy
            qz[sl] = vz
            if it != _ITERS - 1:
                qxr[sl] = _bf(vx)
                qyr[sl] = _bf(vy)
                qzr[sl] = _bf(vz)
                a2s[sl] = (vx * vx + vy * vy) + vz * vz
            return 0

        lax.fori_loop(0, _QPER // _L, blend, 0)

    def out_prep(u, _):
        base = jnp.full((_L,), u * 48, dtype=jnp.int32) + iota3
        sl = pl.ds(u * _L, _L)
        plsc.store_scatter(qbuf, [base], qx[sl])
        plsc.store_scatter(qbuf, [base + 1], qy[sl])
        plsc.store_scatter(qbuf, [base + 2], qz[sl])
        return 0

    lax.fori_loop(0, _QPER // _L, out_prep, 0)
    pltpu.sync_copy(qbuf, out_ref.at[b, pl.ds(qoff, _QPER * 3)])


@jax.jit
def kernel(pred, partial):
    pred2 = pred.reshape(_B, _N * 3)
    part2 = partial.reshape(_B, _M * 3)
    mesh = plsc.VectorSubcoreMesh(core_axis_name="c", subcore_axis_name="s")
    f = pl.kernel(
        _sc_body,
        out_type=jax.ShapeDtypeStruct((_B, _N * 3), jnp.float32),
        mesh=mesh,
        compiler_params=pltpu.CompilerParams(needs_layout_passes=False),
        scratch_types=[
            pltpu.VMEM((_QPER * 3,), jnp.float32),
            pltpu.VMEM((_M * 3,), jnp.float32),
            pltpu.VMEM((_M,), jnp.float32),
            pltpu.VMEM((_M,), jnp.float32),
            pltpu.VMEM((_M,), jnp.float32),
            pltpu.VMEM((_M,), jnp.float32),
            pltpu.VMEM((_M,), jnp.float32),
            pltpu.VMEM((_M,), jnp.float32),
            pltpu.VMEM((_M,), jnp.float32),
            pltpu.VMEM((_QPER,), jnp.float32),
            pltpu.VMEM((_QPER,), jnp.float32),
            pltpu.VMEM((_QPER,), jnp.float32),
            pltpu.VMEM((_QPER,), jnp.float32),
            pltpu.VMEM((_QPER,), jnp.float32),
            pltpu.VMEM((_QPER,), jnp.float32),
            pltpu.VMEM((_QPER,), jnp.float32),
            pltpu.VMEM((_QPER,), jnp.float32),
            pltpu.VMEM((_QPER,), jnp.float32),
            pltpu.VMEM((_QPER,), jnp.float32),
            pltpu.VMEM((_QPER,), jnp.float32),
            pltpu.VMEM((_QPER,), jnp.int32),
            pltpu.VMEM((_L,), jnp.float32),
            pltpu.VMEM((4, _L), jnp.float32),
            pltpu.VMEM_SHARED((_NSUB, _L), jnp.float32),
        ],
    )
    out = f(pred2, part2)
    return out.reshape(_B, _N, 3)


# speedup vs baseline: 6.4137x; 1.0904x over previous
import functools

import jax
import jax.numpy as jnp
import numpy as np
from jax import lax
from jax.experimental import pallas as pl
from jax.experimental.pallas import tpu as pltpu
from jax.experimental.pallas import tpu_sc as plsc
from jax.sharding import Mesh, PartitionSpec as P

_ALPHA = 0.1
_ITERS = 4
_B = 8
_N = 4096
_M = 2048
_L = 16
_NSUB = 16
_NCORE = 2

_GDN = jax.lax.GatherDimensionNumbers(
    offset_dims=(), collapsed_slice_dims=(0,), start_index_map=(0,))


def _lane_bcast(v, j):
    idx = jnp.full((_L,), j, dtype=jnp.int32)
    return lax.gather(v, idx[:, None], dimension_numbers=_GDN,
                      slice_sizes=(1,),
                      mode=lax.GatherScatterMode.PROMISE_IN_BOUNDS)


def _perm(v, idx):
    return lax.gather(v, idx[:, None], dimension_numbers=_GDN,
                      slice_sizes=(1,),
                      mode=lax.GatherScatterMode.PROMISE_IN_BOUNDS)


def _bf(x):
    i = lax.bitcast_convert_type(x, jnp.int32)
    i = i + jnp.int32(0x7FFF) + ((i >> 16) & 1)
    i = i & jnp.int32(-0x10000)
    return lax.bitcast_convert_type(i, jnp.float32)


def _sqrt16(x):
    i = lax.bitcast_convert_type(x, jnp.int32)
    i = jnp.int32(0x5F3759DF) - (i >> 1)
    r = lax.bitcast_convert_type(i, jnp.float32)
    for _ in range(3):
        r = r * (1.5 - 0.5 * x * r * r)
    return x * r


def _tree_min(v, perms):
    for p in perms:
        v = jnp.minimum(v, _perm(v, p))
    return v


def _sc_body(nb, pred_ref, part_ref, out_ref,
             qbuf, kbuf, kx, ky, kz, kxm, kym, kzm, b2,
             qx, qy, qz, qxr, qyr, qzr, a2s,
             nx, ny, nz, nd, bix, stage, grp, shared):
    perb = (_NSUB * _NCORE) // nb
    qper = _N // perb
    ntile = qper // _L
    c = lax.axis_index("c")
    s = lax.axis_index("s")
    b = c * (nb // 2) + s // perb
    qpart = s % perb
    qoff = qpart * (qper * 3)

    iota = lax.iota(jnp.int32, _L)
    iota3 = iota * 3
    perms = [iota ^ (1 << k) for k in range(4)]
    lanesel = [iota == j for j in range(_L)]

    pltpu.sync_copy(part_ref.at[b], kbuf)

    def key_prep(i, _):
        base = jnp.full((_L,), i * 48, dtype=jnp.int32) + iota3
        vx = plsc.load_gather(kbuf, [base])
        vy = plsc.load_gather(kbuf, [base + 1])
        vz = plsc.load_gather(kbuf, [base + 2])
        sl = pl.ds(i * _L, _L)
        kx[sl] = vx
        ky[sl] = vy
        kz[sl] = vz
        kxm[sl] = -2.0 * _bf(vx)
        kym[sl] = -2.0 * _bf(vy)
        kzm[sl] = -2.0 * _bf(vz)
        b2[sl] = (vx * vx + vy * vy) + vz * vz
        return 0

    lax.fori_loop(0, _M // _L, key_prep, 0)

    pltpu.sync_copy(pred_ref.at[b, pl.ds(qoff, qper * 3)], qbuf)

    def q_prep(i, _):
        base = jnp.full((_L,), i * 48, dtype=jnp.int32) + iota3
        sl = pl.ds(i * _L, _L)
        vx = plsc.load_gather(qbuf, [base])
        vy = plsc.load_gather(qbuf, [base + 1])
        vz = plsc.load_gather(qbuf, [base + 2])
        qx[sl] = vx
        qy[sl] = vy
        qz[sl] = vz
        qxr[sl] = _bf(vx)
        qyr[sl] = _bf(vy)
        qzr[sl] = _bf(vz)
        a2s[sl] = (vx * vx + vy * vy) + vz * vz
        return 0

    lax.fori_loop(0, qper // _L, q_prep, 0)

    for it in range(_ITERS):
        def scan_tile(qt, _):
            sl = pl.ds(qt * _L, _L)
            qxv = qxr[sl]
            qyv = qyr[sl]
            qzv = qzr[sl]
            a2v = a2s[sl]
            res_d2 = jnp.zeros((_L,), jnp.float32)
            res_bi = jnp.zeros((_L,), jnp.int32)
            for pair in range(_L // 2):
                j0, j1 = 2 * pair, 2 * pair + 1
                x0 = _lane_bcast(qxv, j0)
                y0 = _lane_bcast(qyv, j0)
                z0 = _lane_bcast(qzv, j0)
                w0 = _lane_bcast(a2v, j0)
                x1 = _lane_bcast(qxv, j1)
                y1 = _lane_bcast(qyv, j1)
                z1 = _lane_bcast(qzv, j1)
                w1 = _lane_bcast(a2v, j1)

                def chunk(ch, carry):
                    bt0, bi0, bt1, bi1, idxv = carry
                    ksl = pl.ds(ch * _L, _L)
                    kvx = kxm[ksl]
                    kvy = kym[ksl]
                    kvz = kzm[ksl]
                    kv2 = b2[ksl]
                    d0 = (w0 + kv2) + ((x0 * kvx + y0 * kvy) + z0 * kvz)
                    m0 = d0 < bt0
                    bt0 = jnp.where(m0, d0, bt0)
                    bi0 = jnp.where(m0, idxv, bi0)
                    d1 = (w1 + kv2) + ((x1 * kvx + y1 * kvy) + z1 * kvz)
                    m1 = d1 < bt1
                    bt1 = jnp.where(m1, d1, bt1)
                    bi1 = jnp.where(m1, idxv, bi1)
                    return bt0, bi0, bt1, bi1, idxv + _L

                init = (jnp.full((_L,), jnp.inf, jnp.float32), iota,
                        jnp.full((_L,), jnp.inf, jnp.float32), iota, iota)
                bt0, bi0, bt1, bi1, _u = plsc.parallel_loop(
                    0, _M // _L, carry=init)(chunk)

                for jq, bt, bi in ((j0, bt0, bi0), (j1, bt1, bi1)):
                    mn = _tree_min(bt, perms)
                    cand = jnp.where(bt == mn, bi, jnp.int32(_M))
                    win = _tree_min(cand, perms)
                    res_d2 = jnp.where(lanesel[jq], mn, res_d2)
                    res_bi = jnp.where(lanesel[jq], win, res_bi)
            nd[sl] = res_d2
            bix[sl] = res_bi
            return 0

        lax.fori_loop(0, ntile, scan_tile, 0)

        def finish_tile(qt, mdv):
            sl = pl.ds(qt * _L, _L)
            biv = bix[sl]
            d = _sqrt16(jnp.maximum(nd[sl], 1e-12))
            nx[sl] = plsc.load_gather(kx, [biv])
            ny[sl] = plsc.load_gather(ky, [biv])
            nz[sl] = plsc.load_gather(kz, [biv])
            nd[sl] = d
            return jnp.maximum(mdv, d)

        mdv = lax.fori_loop(0, ntile, finish_tile,
                            jnp.zeros((_L,), jnp.float32))

        stage[...] = mdv
        pltpu.sync_copy(stage, shared.at[s])
        plsc.subcore_barrier()
        gb = (s // perb) * perb
        pltpu.sync_copy(shared.at[pl.ds(gb, perb)], grp)
        mall = grp[0]
        for r in range(1, perb):
            mall = jnp.maximum(mall, grp[r])
        dmax = -_tree_min(-mall, perms)
        plsc.subcore_barrier()
        denom = dmax + 1e-6

        def blend(u, _):
            sl = pl.ds(u * _L, _L)
            alpha = _ALPHA * (2.0 - nd[sl] / denom)
            vx = qx[sl]
            vy = qy[sl]
            vz = qz[sl]
            vx = vx + alpha * (nx[sl] - vx)
            vy = vy + alpha * (ny[sl] - vy)
            vz = vz + alpha * (nz[sl] - vz)
            qx[sl] = vx
            qy[sl] = vy
            qz[sl] = vz
            if it != _ITERS - 1:
                qxr[sl] = _bf(vx)
                qyr[sl] = _bf(vy)
                qzr[sl] = _bf(vz)
                a2s[sl] = (vx * vx + vy * vy) + vz * vz
            return 0

        lax.fori_loop(0, qper // _L, blend, 0)

    def out_prep(u, _):
        base = jnp.full((_L,), u * 48, dtype=jnp.int32) + iota3
        sl = pl.ds(u * _L, _L)
        plsc.store_scatter(qbuf, [base], qx[sl])
        plsc.store_scatter(qbuf, [base + 1], qy[sl])
        plsc.store_scatter(qbuf, [base + 2], qz[sl])
        return 0

    lax.fori_loop(0, qper // _L, out_prep, 0)
    pltpu.sync_copy(qbuf, out_ref.at[b, pl.ds(qoff, qper * 3)])


@functools.lru_cache(maxsize=None)
def _make_sc_kernel(nb):
    perb = (_NSUB * _NCORE) // nb
    qper = _N // perb
    mesh = plsc.VectorSubcoreMesh(core_axis_name="c", subcore_axis_name="s")
    return pl.kernel(
        functools.partial(_sc_body, nb),
        out_type=jax.ShapeDtypeStruct((nb, _N * 3), jnp.float32),
        mesh=mesh,
        compiler_params=pltpu.CompilerParams(needs_layout_passes=False),
        scratch_types=[
            pltpu.VMEM((qper * 3,), jnp.float32),
            pltpu.VMEM((_M * 3,), jnp.float32),
            pltpu.VMEM((_M,), jnp.float32),
            pltpu.VMEM((_M,), jnp.float32),
            pltpu.VMEM((_M,), jnp.float32),
            pltpu.VMEM((_M,), jnp.float32),
            pltpu.VMEM((_M,), jnp.float32),
            pltpu.VMEM((_M,), jnp.float32),
            pltpu.VMEM((_M,), jnp.float32),
            pltpu.VMEM((qper,), jnp.float32),
            pltpu.VMEM((qper,), jnp.float32),
            pltpu.VMEM((qper,), jnp.float32),
            pltpu.VMEM((qper,), jnp.float32),
            pltpu.VMEM((qper,), jnp.float32),
            pltpu.VMEM((qper,), jnp.float32),
            pltpu.VMEM((qper,), jnp.float32),
            pltpu.VMEM((qper,), jnp.float32),
            pltpu.VMEM((qper,), jnp.float32),
            pltpu.VMEM((qper,), jnp.float32),
            pltpu.VMEM((qper,), jnp.float32),
            pltpu.VMEM((qper,), jnp.int32),
            pltpu.VMEM((_L,), jnp.float32),
            pltpu.VMEM((perb, _L), jnp.float32),
            pltpu.VMEM_SHARED((_NSUB, _L), jnp.float32),
        ],
    )


def kernel(pred, partial):
    pred2 = pred.reshape(_B, _N * 3)
    part2 = partial.reshape(_B, _M * 3)
    devs = jax.devices()
    ndev = 2 if len(devs) >= 2 and _B % 2 == 0 else 1
    if ndev == 2:
        mesh = Mesh(np.array(devs[:2]), ("d",))
        f = jax.shard_map(
            _make_sc_kernel(_B // 2), mesh=mesh,
            in_specs=(P("d"), P("d")), out_specs=P("d"))
        out = f(pred2, part2)
    else:
        out = _make_sc_kernel(_B)(pred2, part2)
    return out.reshape(_B, _N, 3)


# speedup vs baseline: 6.8972x; 1.0754x over previous
import functools

import jax
import jax.numpy as jnp
import numpy as np
from jax import lax
from jax.experimental import pallas as pl
from jax.experimental.pallas import tpu as pltpu
from jax.experimental.pallas import tpu_sc as plsc
from jax.sharding import Mesh, PartitionSpec as P

_ALPHA = 0.1
_ITERS = 4
_B = 8
_N = 4096
_M = 2048
_L = 16
_NSUB = 16
_NCORE = 2

_GDN = jax.lax.GatherDimensionNumbers(
    offset_dims=(), collapsed_slice_dims=(0,), start_index_map=(0,))


def _lane_bcast(v, j):
    idx = jnp.full((_L,), j, dtype=jnp.int32)
    return lax.gather(v, idx[:, None], dimension_numbers=_GDN,
                      slice_sizes=(1,),
                      mode=lax.GatherScatterMode.PROMISE_IN_BOUNDS)


def _perm(v, idx):
    return lax.gather(v, idx[:, None], dimension_numbers=_GDN,
                      slice_sizes=(1,),
                      mode=lax.GatherScatterMode.PROMISE_IN_BOUNDS)


def _bf(x):
    i = lax.bitcast_convert_type(x, jnp.int32)
    i = i + jnp.int32(0x7FFF) + ((i >> 16) & 1)
    i = i & jnp.int32(-0x10000)
    return lax.bitcast_convert_type(i, jnp.float32)


def _sqrt16(x):
    i = lax.bitcast_convert_type(x, jnp.int32)
    i = jnp.int32(0x5F3759DF) - (i >> 1)
    r = lax.bitcast_convert_type(i, jnp.float32)
    for _ in range(3):
        r = r * (1.5 - 0.5 * x * r * r)
    return x * r


def _tree_min(v, perms):
    for p in perms:
        v = jnp.minimum(v, _perm(v, p))
    return v


def _sc_body(nb, pred_ref, part_ref, out_ref,
             qbuf, kbuf, kx, ky, kz, kxm, kym, kzm, b2,
             qx, qy, qz, qxr, qyr, qzr, a2s,
             nx, ny, nz, nd, bix, stage, grp, shared):
    perb = (_NSUB * _NCORE) // nb
    qper = _N // perb
    ntile = qper // _L
    c = lax.axis_index("c")
    s = lax.axis_index("s")
    b = c * (nb // 2) + s // perb
    qpart = s % perb
    qoff = qpart * (qper * 3)

    iota = lax.iota(jnp.int32, _L)
    iota3 = iota * 3
    perms = [iota ^ (1 << k) for k in range(4)]
    lanesel = [iota == j for j in range(_L)]

    pltpu.sync_copy(part_ref.at[b], kbuf)

    def key_prep(i, _):
        base = jnp.full((_L,), i * 48, dtype=jnp.int32) + iota3
        vx = plsc.load_gather(kbuf, [base])
        vy = plsc.load_gather(kbuf, [base + 1])
        vz = plsc.load_gather(kbuf, [base + 2])
        sl = pl.ds(i * _L, _L)
        kx[sl] = vx
        ky[sl] = vy
        kz[sl] = vz
        kxm[sl] = -2.0 * _bf(vx)
        kym[sl] = -2.0 * _bf(vy)
        kzm[sl] = -2.0 * _bf(vz)
        b2[sl] = (vx * vx + vy * vy) + vz * vz
        return 0

    lax.fori_loop(0, _M // _L, key_prep, 0)

    pltpu.sync_copy(pred_ref.at[b, pl.ds(qoff, qper * 3)], qbuf)

    def q_prep(i, _):
        base = jnp.full((_L,), i * 48, dtype=jnp.int32) + iota3
        sl = pl.ds(i * _L, _L)
        vx = plsc.load_gather(qbuf, [base])
        vy = plsc.load_gather(qbuf, [base + 1])
        vz = plsc.load_gather(qbuf, [base + 2])
        qx[sl] = vx
        qy[sl] = vy
        qz[sl] = vz
        qxr[sl] = _bf(vx)
        qyr[sl] = _bf(vy)
        qzr[sl] = _bf(vz)
        a2s[sl] = (vx * vx + vy * vy) + vz * vz
        return 0

    lax.fori_loop(0, qper // _L, q_prep, 0)

    for it in range(_ITERS):
        def scan_tile(qt, _):
            sl = pl.ds(qt * _L, _L)
            qxv = qxr[sl]
            qyv = qyr[sl]
            qzv = qzr[sl]
            a2v = a2s[sl]
            res_d2 = jnp.zeros((_L,), jnp.float32)
            res_bi = jnp.zeros((_L,), jnp.int32)
            for pair in range(_L // 2):
                j0, j1 = 2 * pair, 2 * pair + 1
                x0 = _lane_bcast(qxv, j0)
                y0 = _lane_bcast(qyv, j0)
                z0 = _lane_bcast(qzv, j0)
                w0 = _lane_bcast(a2v, j0)
                x1 = _lane_bcast(qxv, j1)
                y1 = _lane_bcast(qyv, j1)
                z1 = _lane_bcast(qzv, j1)
                w1 = _lane_bcast(a2v, j1)

                def chunk(ch, carry):
                    bt0, bi0, bt1, bi1, idxv = carry
                    ksl = pl.ds(ch * _L, _L)
                    kvx = kxm[ksl]
                    kvy = kym[ksl]
                    kvz = kzm[ksl]
                    kv2 = b2[ksl]
                    d0 = (w0 + kv2) + ((x0 * kvx + y0 * kvy) + z0 * kvz)
                    m0 = d0 < bt0
                    bt0 = jnp.where(m0, d0, bt0)
                    bi0 = jnp.where(m0, idxv, bi0)
                    d1 = (w1 + kv2) + ((x1 * kvx + y1 * kvy) + z1 * kvz)
                    m1 = d1 < bt1
                    bt1 = jnp.where(m1, d1, bt1)
                    bi1 = jnp.where(m1, idxv, bi1)
                    return bt0, bi0, bt1, bi1, idxv + _L

                init = (jnp.full((_L,), jnp.inf, jnp.float32), iota,
                        jnp.full((_L,), jnp.inf, jnp.float32), iota, iota)
                bt0, bi0, bt1, bi1, _u = plsc.parallel_loop(
                    0, _M // _L, carry=init)(chunk)

                for jq, bt, bi in ((j0, bt0, bi0), (j1, bt1, bi1)):
                    mn = _tree_min(bt, perms)
                    cand = jnp.where(bt == mn, bi, jnp.int32(_M))
                    win = _tree_min(cand, perms)
                    res_d2 = jnp.where(lanesel[jq], mn, res_d2)
                    res_bi = jnp.where(lanesel[jq], win, res_bi)
            nd[sl] = res_d2
            bix[sl] = res_bi
            return 0

        lax.fori_loop(0, ntile, scan_tile, 0)

        def finish_tile(qt, mdv):
            sl = pl.ds(qt * _L, _L)
            biv = bix[sl]
            d = _sqrt16(jnp.maximum(nd[sl], 1e-12))
            nx[sl] = plsc.load_gather(kx, [biv])
            ny[sl] = plsc.load_gather(ky, [biv])
            nz[sl] = plsc.load_gather(kz, [biv])
            nd[sl] = d
            return jnp.maximum(mdv, d)

        mdv = lax.fori_loop(0, ntile, finish_tile,
                            jnp.zeros((_L,), jnp.float32))

        stage[...] = mdv
        pltpu.sync_copy(stage, shared.at[s])
        plsc.subcore_barrier()
        gb = (s // perb) * perb
        pltpu.sync_copy(shared.at[pl.ds(gb, perb)], grp)
        mall = grp[0]
        for r in range(1, perb):
            mall = jnp.maximum(mall, grp[r])
        dmax = -_tree_min(-mall, perms)
        plsc.subcore_barrier()
        denom = dmax + 1e-6

        def blend(u, _):
            sl = pl.ds(u * _L, _L)
            alpha = _ALPHA * (2.0 - nd[sl] / denom)
            vx = qx[sl]
            vy = qy[sl]
            vz = qz[sl]
            vx = vx + alpha * (nx[sl] - vx)
            vy = vy + alpha * (ny[sl] - vy)
            vz = vz + alpha * (nz[sl] - vz)
            qx[sl] = vx
            qy[sl] = vy
            qz[sl] = vz
            if it != _ITERS - 1:
                qxr[sl] = _bf(vx)
                qyr[sl] = _bf(vy)
                qzr[sl] = _bf(vz)
                a2s[sl] = (vx * vx + vy * vy) + vz * vz
            return 0

        lax.fori_loop(0, qper // _L, blend, 0)

    def out_prep(u, _):
        base = jnp.full((_L,), u * 48, dtype=jnp.int32) + iota3
        sl = pl.ds(u * _L, _L)
        plsc.store_scatter(qbuf, [base], qx[sl])
        plsc.store_scatter(qbuf, [base + 1], qy[sl])
        plsc.store_scatter(qbuf, [base + 2], qz[sl])
        return 0

    lax.fori_loop(0, qper // _L, out_prep, 0)
    pltpu.sync_copy(qbuf, out_ref.at[b, pl.ds(qoff, qper * 3)])


@functools.lru_cache(maxsize=None)
def _make_sc_kernel(nb):
    perb = (_NSUB * _NCORE) // nb
    qper = _N // perb
    mesh = plsc.VectorSubcoreMesh(core_axis_name="c", subcore_axis_name="s")
    return pl.kernel(
        functools.partial(_sc_body, nb),
        out_type=jax.ShapeDtypeStruct((nb, _N * 3), jnp.float32),
        mesh=mesh,
        compiler_params=pltpu.CompilerParams(needs_layout_passes=False),
        scratch_types=[
            pltpu.VMEM((qper * 3,), jnp.float32),
            pltpu.VMEM((_M * 3,), jnp.float32),
            pltpu.VMEM((_M,), jnp.float32),
            pltpu.VMEM((_M,), jnp.float32),
            pltpu.VMEM((_M,), jnp.float32),
            pltpu.VMEM((_M,), jnp.float32),
            pltpu.VMEM((_M,), jnp.float32),
            pltpu.VMEM((_M,), jnp.float32),
            pltpu.VMEM((_M,), jnp.float32),
            pltpu.VMEM((qper,), jnp.float32),
            pltpu.VMEM((qper,), jnp.float32),
            pltpu.VMEM((qper,), jnp.float32),
            pltpu.VMEM((qper,), jnp.float32),
            pltpu.VMEM((qper,), jnp.float32),
            pltpu.VMEM((qper,), jnp.float32),
            pltpu.VMEM((qper,), jnp.float32),
            pltpu.VMEM((qper,), jnp.float32),
            pltpu.VMEM((qper,), jnp.float32),
            pltpu.VMEM((qper,), jnp.float32),
            pltpu.VMEM((qper,), jnp.float32),
            pltpu.VMEM((qper,), jnp.int32),
            pltpu.VMEM((_L,), jnp.float32),
            pltpu.VMEM((perb, _L), jnp.float32),
            pltpu.VMEM_SHARED((_NSUB, _L), jnp.float32),
        ],
    )


_KCHUNK = 128


def _tc_body(pred_ref, part_ref, out_ref, ktb_ref, b2_ref):
    p3 = pred_ref[0]
    kt = part_ref[0]
    n = p3.shape[0]
    m = kt.shape[1]
    nchunks = m // _KCHUNK

    def bfr(x):
        return x.astype(jnp.bfloat16).astype(jnp.float32)

    ktb_ref[...] = bfr(kt)
    b2_ref[...] = kt[0:1, :] ** 2 + kt[1:2, :] ** 2 + kt[2:3, :] ** 2

    qx = p3[:, 0:1]
    qy = p3[:, 1:2]
    qz = p3[:, 2:3]

    lane = jax.lax.broadcasted_iota(jnp.int32, (n, _KCHUNK), 1)

    for _ in range(_ITERS):
        qxb, qyb, qzb = bfr(qx), bfr(qy), bfr(qz)
        a2 = qx * qx + qy * qy + qz * qz

        def chunk_step(cc, carry):
            best, bx, by, bz = carry
            sl = pl.ds(cc * _KCHUNK, _KCHUNK)
            kxb = ktb_ref[0:1, sl]
            kyb = ktb_ref[1:2, sl]
            kzb = ktb_ref[2:3, sl]
            kxE = part_ref[0, 0:1, sl]
            kyE = part_ref[0, 1:2, sl]
            kzE = part_ref[0, 2:3, sl]
            dot = qxb * kxb + qyb * kyb + qzb * kzb
            d2 = (a2 + b2_ref[0:1, sl]) - 2.0 * dot
            msk = d2 < best
            best = jnp.where(msk, d2, best)
            bx = jnp.where(msk, jnp.broadcast_to(kxE, (n, _KCHUNK)), bx)
            by = jnp.where(msk, jnp.broadcast_to(kyE, (n, _KCHUNK)), by)
            bz = jnp.where(msk, jnp.broadcast_to(kzE, (n, _KCHUNK)), bz)
            return best, bx, by, bz

        init = (jnp.full((n, _KCHUNK), jnp.inf, jnp.float32),
                jnp.zeros((n, _KCHUNK), jnp.float32),
                jnp.zeros((n, _KCHUNK), jnp.float32),
                jnp.zeros((n, _KCHUNK), jnp.float32))
        best, bx, by, bz = jax.lax.fori_loop(0, nchunks, chunk_step, init)

        bmin = jnp.min(best, axis=1, keepdims=True)
        eq = best == bmin
        li = jnp.min(jnp.where(eq, lane, _KCHUNK), axis=1, keepdims=True)
        pick = lane == li
        cx = jnp.sum(jnp.where(pick, bx, 0.0), axis=1, keepdims=True)
        cy = jnp.sum(jnp.where(pick, by, 0.0), axis=1, keepdims=True)
        cz = jnp.sum(jnp.where(pick, bz, 0.0), axis=1, keepdims=True)

        d = jnp.sqrt(jnp.maximum(bmin, 1e-12))
        dmax = jnp.max(d)
        alpha = _ALPHA * (2.0 - d / (dmax + 1e-6))
        qx = qx + alpha * (cx - qx)
        qy = qy + alpha * (cy - qy)
        qz = qz + alpha * (cz - qz)

    out_ref[0] = jnp.concatenate([qx, qy, qz], axis=1)


def _tc_call(pred, partial):
    b, n, _ = pred.shape
    m = partial.shape[1]
    part_t = jnp.swapaxes(partial, 1, 2)
    return pl.pallas_call(
        _tc_body,
        grid=(b,),
        in_specs=[
            pl.BlockSpec((1, n, 3), lambda i: (i, 0, 0)),
            pl.BlockSpec((1, 3, m), lambda i: (i, 0, 0)),
        ],
        out_specs=pl.BlockSpec((1, n, 3), lambda i: (i, 0, 0)),
        out_shape=jax.ShapeDtypeStruct((b, n, 3), jnp.float32),
        scratch_shapes=[
            pltpu.VMEM((3, m), jnp.float32),
            pltpu.VMEM((1, m), jnp.float32),
        ],
    )(pred, part_t)


_SCB = 4


def kernel(pred, partial):
    pred2 = pred.reshape(_B, _N * 3)
    part2 = partial.reshape(_B, _M * 3)
    sc_out = _make_sc_kernel(_SCB)(pred2[:_SCB], part2[:_SCB])
    tc_out = _tc_call(pred[_SCB:], partial[_SCB:])
    return jnp.concatenate([sc_out.reshape(_SCB, _N, 3), tc_out], axis=0)


# speedup vs baseline: 7.1149x; 1.0316x over previous
import functools

import jax
import jax.numpy as jnp
import numpy as np
from jax import lax
from jax.experimental import pallas as pl
from jax.experimental.pallas import tpu as pltpu
from jax.experimental.pallas import tpu_sc as plsc
from jax.sharding import Mesh, PartitionSpec as P

_ALPHA = 0.1
_ITERS = 4
_B = 8
_N = 4096
_M = 2048
_L = 16
_NSUB = 16
_NCORE = 2

_GDN = jax.lax.GatherDimensionNumbers(
    offset_dims=(), collapsed_slice_dims=(0,), start_index_map=(0,))


def _lane_bcast(v, j):
    idx = jnp.full((_L,), j, dtype=jnp.int32)
    return lax.gather(v, idx[:, None], dimension_numbers=_GDN,
                      slice_sizes=(1,),
                      mode=lax.GatherScatterMode.PROMISE_IN_BOUNDS)


def _perm(v, idx):
    return lax.gather(v, idx[:, None], dimension_numbers=_GDN,
                      slice_sizes=(1,),
                      mode=lax.GatherScatterMode.PROMISE_IN_BOUNDS)


def _bf(x):
    i = lax.bitcast_convert_type(x, jnp.int32)
    i = i + jnp.int32(0x7FFF) + ((i >> 16) & 1)
    i = i & jnp.int32(-0x10000)
    return lax.bitcast_convert_type(i, jnp.float32)


def _sqrt16(x):
    i = lax.bitcast_convert_type(x, jnp.int32)
    i = jnp.int32(0x5F3759DF) - (i >> 1)
    r = lax.bitcast_convert_type(i, jnp.float32)
    for _ in range(3):
        r = r * (1.5 - 0.5 * x * r * r)
    return x * r


def _tree_min(v, perms):
    for p in perms:
        v = jnp.minimum(v, _perm(v, p))
    return v


def _sc_body(nb, pred_ref, part_ref, out_ref,
             qbuf, kbuf, kx, ky, kz, kxm, kym, kzm, b2,
             qx, qy, qz, qxr, qyr, qzr, a2s,
             nx, ny, nz, nd, bix, stage, grp, shared):
    perb = (_NSUB * _NCORE) // nb
    qper = _N // perb
    ntile = qper // _L
    c = lax.axis_index("c")
    s = lax.axis_index("s")
    b = c * (nb // 2) + s // perb
    qpart = s % perb
    qoff = qpart * (qper * 3)

    iota = lax.iota(jnp.int32, _L)
    iota3 = iota * 3
    perms = [iota ^ (1 << k) for k in range(4)]
    lanesel = [iota == j for j in range(_L)]

    pltpu.sync_copy(part_ref.at[b], kbuf)

    def key_prep(i, _):
        base = jnp.full((_L,), i * 48, dtype=jnp.int32) + iota3
        vx = plsc.load_gather(kbuf, [base])
        vy = plsc.load_gather(kbuf, [base + 1])
        vz = plsc.load_gather(kbuf, [base + 2])
        sl = pl.ds(i * _L, _L)
        kx[sl] = vx
        ky[sl] = vy
        kz[sl] = vz
        kxm[sl] = -2.0 * _bf(vx)
        kym[sl] = -2.0 * _bf(vy)
        kzm[sl] = -2.0 * _bf(vz)
        b2[sl] = (vx * vx + vy * vy) + vz * vz
        return 0

    lax.fori_loop(0, _M // _L, key_prep, 0)

    pltpu.sync_copy(pred_ref.at[b, pl.ds(qoff, qper * 3)], qbuf)

    def q_prep(i, _):
        base = jnp.full((_L,), i * 48, dtype=jnp.int32) + iota3
        sl = pl.ds(i * _L, _L)
        vx = plsc.load_gather(qbuf, [base])
        vy = plsc.load_gather(qbuf, [base + 1])
        vz = plsc.load_gather(qbuf, [base + 2])
        qx[sl] = vx
        qy[sl] = vy
        qz[sl] = vz
        qxr[sl] = _bf(vx)
        qyr[sl] = _bf(vy)
        qzr[sl] = _bf(vz)
        a2s[sl] = (vx * vx + vy * vy) + vz * vz
        return 0

    lax.fori_loop(0, qper // _L, q_prep, 0)

    for it in range(_ITERS):
        def scan_tile(qt, _):
            sl = pl.ds(qt * _L, _L)
            qxv = qxr[sl]
            qyv = qyr[sl]
            qzv = qzr[sl]
            a2v = a2s[sl]
            res_d2 = jnp.zeros((_L,), jnp.float32)
            res_bi = jnp.zeros((_L,), jnp.int32)
            for pair in range(_L // 2):
                j0, j1 = 2 * pair, 2 * pair + 1
                x0 = _lane_bcast(qxv, j0)
                y0 = _lane_bcast(qyv, j0)
                z0 = _lane_bcast(qzv, j0)
                w0 = _lane_bcast(a2v, j0)
                x1 = _lane_bcast(qxv, j1)
                y1 = _lane_bcast(qyv, j1)
                z1 = _lane_bcast(qzv, j1)
                w1 = _lane_bcast(a2v, j1)

                def chunk(ch, carry):
                    bt0, bi0, bt1, bi1, idxv = carry
                    ksl = pl.ds(ch * _L, _L)
                    kvx = kxm[ksl]
                    kvy = kym[ksl]
                    kvz = kzm[ksl]
                    kv2 = b2[ksl]
                    d0 = (w0 + kv2) + ((x0 * kvx + y0 * kvy) + z0 * kvz)
                    m0 = d0 < bt0
                    bt0 = jnp.where(m0, d0, bt0)
                    bi0 = jnp.where(m0, idxv, bi0)
                    d1 = (w1 + kv2) + ((x1 * kvx + y1 * kvy) + z1 * kvz)
                    m1 = d1 < bt1
                    bt1 = jnp.where(m1, d1, bt1)
                    bi1 = jnp.where(m1, idxv, bi1)
                    return bt0, bi0, bt1, bi1, idxv + _L

                init = (jnp.full((_L,), jnp.inf, jnp.float32), iota,
                        jnp.full((_L,), jnp.inf, jnp.float32), iota, iota)
                bt0, bi0, bt1, bi1, _u = plsc.parallel_loop(
                    0, _M // _L, carry=init)(chunk)

                for jq, bt, bi in ((j0, bt0, bi0), (j1, bt1, bi1)):
                    mn = _tree_min(bt, perms)
                    cand = jnp.where(bt == mn, bi, jnp.int32(_M))
                    win = _tree_min(cand, perms)
                    res_d2 = jnp.where(lanesel[jq], mn, res_d2)
                    res_bi = jnp.where(lanesel[jq], win, res_bi)
            nd[sl] = res_d2
            bix[sl] = res_bi
            return 0

        lax.fori_loop(0, ntile, scan_tile, 0)

        def finish_tile(qt, mdv):
            sl = pl.ds(qt * _L, _L)
            biv = bix[sl]
            d = _sqrt16(jnp.maximum(nd[sl], 1e-12))
            nx[sl] = plsc.load_gather(kx, [biv])
            ny[sl] = plsc.load_gather(ky, [biv])
            nz[sl] = plsc.load_gather(kz, [biv])
            nd[sl] = d
            return jnp.maximum(mdv, d)

        mdv = lax.fori_loop(0, ntile, finish_tile,
                            jnp.zeros((_L,), jnp.float32))

        stage[...] = mdv
        pltpu.sync_copy(stage, shared.at[s])
        plsc.subcore_barrier()
        gb = (s // perb) * perb
        pltpu.sync_copy(shared.at[pl.ds(gb, perb)], grp)
        mall = grp[0]
        for r in range(1, perb):
            mall = jnp.maximum(mall, grp[r])
        dmax = -_tree_min(-mall, perms)
        plsc.subcore_barrier()
        denom = dmax + 1e-6

        def blend(u, _):
            sl = pl.ds(u * _L, _L)
            alpha = _ALPHA * (2.0 - nd[sl] / denom)
            vx = qx[sl]
            vy = qy[sl]
            vz = qz[sl]
            vx = vx + alpha * (nx[sl] - vx)
            vy = vy + alpha * (ny[sl] - vy)
            vz = vz + alpha * (nz[sl] - vz)
            qx[sl] = vx
            qy[sl] = vy
            qz[sl] = vz
            if it != _ITERS - 1:
                qxr[sl] = _bf(vx)
                qyr[sl] = _bf(vy)
                qzr[sl] = _bf(vz)
                a2s[sl] = (vx * vx + vy * vy) + vz * vz
            return 0

        lax.fori_loop(0, qper // _L, blend, 0)

    def out_prep(u, _):
        base = jnp.full((_L,), u * 48, dtype=jnp.int32) + iota3
        sl = pl.ds(u * _L, _L)
        plsc.store_scatter(qbuf, [base], qx[sl])
        plsc.store_scatter(qbuf, [base + 1], qy[sl])
        plsc.store_scatter(qbuf, [base + 2], qz[sl])
        return 0

    lax.fori_loop(0, qper // _L, out_prep, 0)
    pltpu.sync_copy(qbuf, out_ref.at[b, pl.ds(qoff, qper * 3)])


@functools.lru_cache(maxsize=None)
def _make_sc_kernel(nb):
    perb = (_NSUB * _NCORE) // nb
    qper = _N // perb
    mesh = plsc.VectorSubcoreMesh(core_axis_name="c", subcore_axis_name="s")
    return pl.kernel(
        functools.partial(_sc_body, nb),
        out_type=jax.ShapeDtypeStruct((nb, _N * 3), jnp.float32),
        mesh=mesh,
        compiler_params=pltpu.CompilerParams(needs_layout_passes=False),
        scratch_types=[
            pltpu.VMEM((qper * 3,), jnp.float32),
            pltpu.VMEM((_M * 3,), jnp.float32),
            pltpu.VMEM((_M,), jnp.float32),
            pltpu.VMEM((_M,), jnp.float32),
            pltpu.VMEM((_M,), jnp.float32),
            pltpu.VMEM((_M,), jnp.float32),
            pltpu.VMEM((_M,), jnp.float32),
            pltpu.VMEM((_M,), jnp.float32),
            pltpu.VMEM((_M,), jnp.float32),
            pltpu.VMEM((qper,), jnp.float32),
            pltpu.VMEM((qper,), jnp.float32),
            pltpu.VMEM((qper,), jnp.float32),
            pltpu.VMEM((qper,), jnp.float32),
            pltpu.VMEM((qper,), jnp.float32),
            pltpu.VMEM((qper,), jnp.float32),
            pltpu.VMEM((qper,), jnp.float32),
            pltpu.VMEM((qper,), jnp.float32),
            pltpu.VMEM((qper,), jnp.float32),
            pltpu.VMEM((qper,), jnp.float32),
            pltpu.VMEM((qper,), jnp.float32),
            pltpu.VMEM((qper,), jnp.int32),
            pltpu.VMEM((_L,), jnp.float32),
            pltpu.VMEM((perb, _L), jnp.float32),
            pltpu.VMEM_SHARED((_NSUB, _L), jnp.float32),
        ],
    )


_KCHUNK = 128


def _tc_body(pred_ref, part_ref, out_ref, ktb_ref, b2_ref):
    p3 = pred_ref[0]
    kt = part_ref[0]
    n = p3.shape[0]
    m = kt.shape[1]
    nchunks = m // _KCHUNK

    def bfr(x):
        return x.astype(jnp.bfloat16).astype(jnp.float32)

    ktb_ref[...] = bfr(kt)
    b2_ref[...] = kt[0:1, :] ** 2 + kt[1:2, :] ** 2 + kt[2:3, :] ** 2

    qx = p3[:, 0:1]
    qy = p3[:, 1:2]
    qz = p3[:, 2:3]

    lane = jax.lax.broadcasted_iota(jnp.int32, (n, _KCHUNK), 1)

    for _ in range(_ITERS):
        qxb, qyb, qzb = bfr(qx), bfr(qy), bfr(qz)
        a2 = qx * qx + qy * qy + qz * qz

        def chunk_step(cc, carry):
            best, bx, by, bz = carry
            sl = pl.ds(cc * _KCHUNK, _KCHUNK)
            kxb = ktb_ref[0:1, sl]
            kyb = ktb_ref[1:2, sl]
            kzb = ktb_ref[2:3, sl]
            kxE = part_ref[0, 0:1, sl]
            kyE = part_ref[0, 1:2, sl]
            kzE = part_ref[0, 2:3, sl]
            dot = qxb * kxb + qyb * kyb + qzb * kzb
            d2 = (a2 + b2_ref[0:1, sl]) - 2.0 * dot
            msk = d2 < best
            best = jnp.where(msk, d2, best)
            bx = jnp.where(msk, jnp.broadcast_to(kxE, (n, _KCHUNK)), bx)
            by = jnp.where(msk, jnp.broadcast_to(kyE, (n, _KCHUNK)), by)
            bz = jnp.where(msk, jnp.broadcast_to(kzE, (n, _KCHUNK)), bz)
            return best, bx, by, bz

        init = (jnp.full((n, _KCHUNK), jnp.inf, jnp.float32),
                jnp.zeros((n, _KCHUNK), jnp.float32),
                jnp.zeros((n, _KCHUNK), jnp.float32),
                jnp.zeros((n, _KCHUNK), jnp.float32))
        best, bx, by, bz = jax.lax.fori_loop(0, nchunks, chunk_step, init)

        bmin = jnp.min(best, axis=1, keepdims=True)
        eq = best == bmin
        li = jnp.min(jnp.where(eq, lane, _KCHUNK), axis=1, keepdims=True)
        pick = lane == li
        cx = jnp.sum(jnp.where(pick, bx, 0.0), axis=1, keepdims=True)
        cy = jnp.sum(jnp.where(pick, by, 0.0), axis=1, keepdims=True)
        cz = jnp.sum(jnp.where(pick, bz, 0.0), axis=1, keepdims=True)

        d = jnp.sqrt(jnp.maximum(bmin, 1e-12))
        dmax = jnp.max(d)
        alpha = _ALPHA * (2.0 - d / (dmax + 1e-6))
        qx = qx + alpha * (cx - qx)
        qy = qy + alpha * (cy - qy)
        qz = qz + alpha * (cz - qz)

    out_ref[0] = jnp.concatenate([qx, qy, qz], axis=1)


def _tc_call(pred, partial):
    b, n, _ = pred.shape
    m = partial.shape[1]
    part_t = jnp.swapaxes(partial, 1, 2)
    return pl.pallas_call(
        _tc_body,
        grid=(b,),
        in_specs=[
            pl.BlockSpec((1, n, 3), lambda i: (i, 0, 0)),
            pl.BlockSpec((1, 3, m), lambda i: (i, 0, 0)),
        ],
        out_specs=pl.BlockSpec((1, n, 3), lambda i: (i, 0, 0)),
        out_shape=jax.ShapeDtypeStruct((b, n, 3), jnp.float32),
        scratch_shapes=[
            pltpu.VMEM((3, m), jnp.float32),
            pltpu.VMEM((1, m), jnp.float32),
        ],
    )(pred, part_t)


_SCB = 4


def _device_kernel(scb, pred, partial):
    b = pred.shape[0]
    pred2 = pred.reshape(b, _N * 3)
    part2 = partial.reshape(b, _M * 3)
    parts = []
    if scb:
        sc_out = _make_sc_kernel(scb)(pred2[:scb], part2[:scb])
        parts.append(sc_out.reshape(scb, _N, 3))
    if b - scb:
        parts.append(_tc_call(pred[scb:], partial[scb:]))
    return jnp.concatenate(parts, axis=0) if len(parts) > 1 else parts[0]


def kernel(pred, partial):
    devs = jax.devices()
    if len(devs) >= 2:
        mesh = Mesh(np.array(devs[:2]), ("d",))
        f = jax.shard_map(
            functools.partial(_device_kernel, _SCB // 2), mesh=mesh,
            in_specs=(P("d"), P("d")), out_specs=P("d"), check_vma=False)
        return f(pred, partial)
    return _device_kernel(_SCB, pred, partial)


# speedup vs baseline: 8.6660x; 1.2180x over previous
import functools

import jax
import jax.numpy as jnp
import numpy as np
from jax import lax
from jax.experimental import pallas as pl
from jax.experimental.pallas import tpu as pltpu
from jax.experimental.pallas import tpu_sc as plsc
from jax.sharding import Mesh, PartitionSpec as P

_ALPHA = 0.1
_ITERS = 4
_B = 8
_N = 4096
_M = 2048
_L = 16
_NSUB = 16
_NCORE = 2

_GDN = jax.lax.GatherDimensionNumbers(
    offset_dims=(), collapsed_slice_dims=(0,), start_index_map=(0,))


def _lane_bcast(v, j):
    idx = jnp.full((_L,), j, dtype=jnp.int32)
    return lax.gather(v, idx[:, None], dimension_numbers=_GDN,
                      slice_sizes=(1,),
                      mode=lax.GatherScatterMode.PROMISE_IN_BOUNDS)


def _perm(v, idx):
    return lax.gather(v, idx[:, None], dimension_numbers=_GDN,
                      slice_sizes=(1,),
                      mode=lax.GatherScatterMode.PROMISE_IN_BOUNDS)


def _bf(x):
    i = lax.bitcast_convert_type(x, jnp.int32)
    i = i + jnp.int32(0x7FFF) + ((i >> 16) & 1)
    i = i & jnp.int32(-0x10000)
    return lax.bitcast_convert_type(i, jnp.float32)


def _sqrt16(x):
    i = lax.bitcast_convert_type(x, jnp.int32)
    i = jnp.int32(0x5F3759DF) - (i >> 1)
    r = lax.bitcast_convert_type(i, jnp.float32)
    for _ in range(3):
        r = r * (1.5 - 0.5 * x * r * r)
    return x * r


def _tree_min(v, perms):
    for p in perms:
        v = jnp.minimum(v, _perm(v, p))
    return v


def _sc_body(nb, pred_ref, part_ref, out_ref,
             qbuf, kbuf, kx, ky, kz, kxm, kym, kzm, b2,
             qx, qy, qz, qxr, qyr, qzr, a2s,
             nx, ny, nz, nd, bix, stage, grp, shared):
    perb = (_NSUB * _NCORE) // nb
    qper = _N // perb
    ntile = qper // _L
    c = lax.axis_index("c")
    s = lax.axis_index("s")
    b = c * (nb // 2) + s // perb
    qpart = s % perb
    qoff = qpart * (qper * 3)

    iota = lax.iota(jnp.int32, _L)
    iota3 = iota * 3
    perms = [iota ^ (1 << k) for k in range(4)]
    lanesel = [iota == j for j in range(_L)]

    pltpu.sync_copy(part_ref.at[b], kbuf)

    def key_prep(i, _):
        base = jnp.full((_L,), i * 48, dtype=jnp.int32) + iota3
        vx = plsc.load_gather(kbuf, [base])
        vy = plsc.load_gather(kbuf, [base + 1])
        vz = plsc.load_gather(kbuf, [base + 2])
        sl = pl.ds(i * _L, _L)
        kx[sl] = vx
        ky[sl] = vy
        kz[sl] = vz
        kxm[sl] = -2.0 * _bf(vx)
        kym[sl] = -2.0 * _bf(vy)
        kzm[sl] = -2.0 * _bf(vz)
        b2[sl] = (vx * vx + vy * vy) + vz * vz
        return 0

    lax.fori_loop(0, _M // _L, key_prep, 0)

    pltpu.sync_copy(pred_ref.at[b, pl.ds(qoff, qper * 3)], qbuf)

    def q_prep(i, _):
        base = jnp.full((_L,), i * 48, dtype=jnp.int32) + iota3
        sl = pl.ds(i * _L, _L)
        vx = plsc.load_gather(qbuf, [base])
        vy = plsc.load_gather(qbuf, [base + 1])
        vz = plsc.load_gather(qbuf, [base + 2])
        qx[sl] = vx
        qy[sl] = vy
        qz[sl] = vz
        qxr[sl] = _bf(vx)
        qyr[sl] = _bf(vy)
        qzr[sl] = _bf(vz)
        a2s[sl] = (vx * vx + vy * vy) + vz * vz
        return 0

    lax.fori_loop(0, qper // _L, q_prep, 0)

    for it in range(_ITERS):
        def scan_tile(qt, _):
            sl = pl.ds(qt * _L, _L)
            qxv = qxr[sl]
            qyv = qyr[sl]
            qzv = qzr[sl]
            a2v = a2s[sl]
            res_d2 = jnp.zeros((_L,), jnp.float32)
            res_bi = jnp.zeros((_L,), jnp.int32)
            for pair in range(_L // 2):
                j0, j1 = 2 * pair, 2 * pair + 1
                x0 = _lane_bcast(qxv, j0)
                y0 = _lane_bcast(qyv, j0)
                z0 = _lane_bcast(qzv, j0)
                w0 = _lane_bcast(a2v, j0)
                x1 = _lane_bcast(qxv, j1)
                y1 = _lane_bcast(qyv, j1)
                z1 = _lane_bcast(qzv, j1)
                w1 = _lane_bcast(a2v, j1)

                def chunk(ch, carry):
                    bt0, bi0, bt1, bi1, idxv = carry
                    ksl = pl.ds(ch * _L, _L)
                    kvx = kxm[ksl]
                    kvy = kym[ksl]
                    kvz = kzm[ksl]
                    kv2 = b2[ksl]
                    d0 = (w0 + kv2) + ((x0 * kvx + y0 * kvy) + z0 * kvz)
                    m0 = d0 < bt0
                    bt0 = jnp.where(m0, d0, bt0)
                    bi0 = jnp.where(m0, idxv, bi0)
                    d1 = (w1 + kv2) + ((x1 * kvx + y1 * kvy) + z1 * kvz)
                    m1 = d1 < bt1
                    bt1 = jnp.where(m1, d1, bt1)
                    bi1 = jnp.where(m1, idxv, bi1)
                    return bt0, bi0, bt1, bi1, idxv + _L

                init = (jnp.full((_L,), jnp.inf, jnp.float32), iota,
                        jnp.full((_L,), jnp.inf, jnp.float32), iota, iota)
                bt0, bi0, bt1, bi1, _u = plsc.parallel_loop(
                    0, _M // _L, carry=init)(chunk)

                for jq, bt, bi in ((j0, bt0, bi0), (j1, bt1, bi1)):
                    mn = _tree_min(bt, perms)
                    cand = jnp.where(bt == mn, bi, jnp.int32(_M))
                    win = _tree_min(cand, perms)
                    res_d2 = jnp.where(lanesel[jq], mn, res_d2)
                    res_bi = jnp.where(lanesel[jq], win, res_bi)
            nd[sl] = res_d2
            bix[sl] = res_bi
            return 0

        lax.fori_loop(0, ntile, scan_tile, 0)

        def finish_tile(qt, mdv):
            sl = pl.ds(qt * _L, _L)
            biv = bix[sl]
            d = _sqrt16(jnp.maximum(nd[sl], 1e-12))
            nx[sl] = plsc.load_gather(kx, [biv])
            ny[sl] = plsc.load_gather(ky, [biv])
            nz[sl] = plsc.load_gather(kz, [biv])
            nd[sl] = d
            return jnp.maximum(mdv, d)

        mdv = lax.fori_loop(0, ntile, finish_tile,
                            jnp.zeros((_L,), jnp.float32))

        stage[...] = mdv
        pltpu.sync_copy(stage, shared.at[s])
        plsc.subcore_barrier()
        gb = (s // perb) * perb
        pltpu.sync_copy(shared.at[pl.ds(gb, perb)], grp)
        mall = grp[0]
        for r in range(1, perb):
            mall = jnp.maximum(mall, grp[r])
        dmax = -_tree_min(-mall, perms)
        plsc.subcore_barrier()
        denom = dmax + 1e-6

        def blend(u, _):
            sl = pl.ds(u * _L, _L)
            alpha = _ALPHA * (2.0 - nd[sl] / denom)
            vx = qx[sl]
            vy = qy[sl]
            vz = qz[sl]
            vx = vx + alpha * (nx[sl] - vx)
            vy = vy + alpha * (ny[sl] - vy)
            vz = vz + alpha * (nz[sl] - vz)
            qx[sl] = vx
            qy[sl] = vy
            qz[sl] = vz
            if it != _ITERS - 1:
                qxr[sl] = _bf(vx)
                qyr[sl] = _bf(vy)
                qzr[sl] = _bf(vz)
                a2s[sl] = (vx * vx + vy * vy) + vz * vz
            return 0

        lax.fori_loop(0, qper // _L, blend, 0)

    def out_prep(u, _):
        base = jnp.full((_L,), u * 48, dtype=jnp.int32) + iota3
        sl = pl.ds(u * _L, _L)
        plsc.store_scatter(qbuf, [base], qx[sl])
        plsc.store_scatter(qbuf, [base + 1], qy[sl])
        plsc.store_scatter(qbuf, [base + 2], qz[sl])
        return 0

    lax.fori_loop(0, qper // _L, out_prep, 0)
    pltpu.sync_copy(qbuf, out_ref.at[b, pl.ds(qoff, qper * 3)])


@functools.lru_cache(maxsize=None)
def _make_sc_kernel(nb):
    perb = (_NSUB * _NCORE) // nb
    qper = _N // perb
    mesh = plsc.VectorSubcoreMesh(core_axis_name="c", subcore_axis_name="s")
    return pl.kernel(
        functools.partial(_sc_body, nb),
        out_type=jax.ShapeDtypeStruct((nb, _N * 3), jnp.float32),
        mesh=mesh,
        compiler_params=pltpu.CompilerParams(needs_layout_passes=False),
        scratch_types=[
            pltpu.VMEM((qper * 3,), jnp.float32),
            pltpu.VMEM((_M * 3,), jnp.float32),
            pltpu.VMEM((_M,), jnp.float32),
            pltpu.VMEM((_M,), jnp.float32),
            pltpu.VMEM((_M,), jnp.float32),
            pltpu.VMEM((_M,), jnp.float32),
            pltpu.VMEM((_M,), jnp.float32),
            pltpu.VMEM((_M,), jnp.float32),
            pltpu.VMEM((_M,), jnp.float32),
            pltpu.VMEM((qper,), jnp.float32),
            pltpu.VMEM((qper,), jnp.float32),
            pltpu.VMEM((qper,), jnp.float32),
            pltpu.VMEM((qper,), jnp.float32),
            pltpu.VMEM((qper,), jnp.float32),
            pltpu.VMEM((qper,), jnp.float32),
            pltpu.VMEM((qper,), jnp.float32),
            pltpu.VMEM((qper,), jnp.float32),
            pltpu.VMEM((qper,), jnp.float32),
            pltpu.VMEM((qper,), jnp.float32),
            pltpu.VMEM((qper,), jnp.float32),
            pltpu.VMEM((qper,), jnp.int32),
            pltpu.VMEM((_L,), jnp.float32),
            pltpu.VMEM((perb, _L), jnp.float32),
            pltpu.VMEM_SHARED((_NSUB, _L), jnp.float32),
        ],
    )


_KCHUNK = 128


def _tc_body(pred_ref, part_ref, out_ref, ktb_ref, b2_ref):
    p3 = pred_ref[0]
    kt = part_ref[0]
    n = p3.shape[0]
    m = kt.shape[1]
    nchunks = m // _KCHUNK

    ktb_ref[...] = kt.astype(jnp.bfloat16)
    b2_ref[...] = kt[0:1, :] ** 2 + kt[1:2, :] ** 2 + kt[2:3, :] ** 2

    qx = p3[:, 0:1]
    qy = p3[:, 1:2]
    qz = p3[:, 2:3]

    lane = jax.lax.broadcasted_iota(jnp.int32, (n, _KCHUNK), 1)

    for _ in range(_ITERS):
        qmat = jnp.concatenate([qx, qy, qz], axis=1).astype(jnp.bfloat16)
        a2 = qx * qx + qy * qy + qz * qz

        def chunk_step(cc, carry):
            best, bx, by, bz = carry
            sl = pl.ds(cc * _KCHUNK, _KCHUNK)
            kxE = part_ref[0, 0:1, sl]
            kyE = part_ref[0, 1:2, sl]
            kzE = part_ref[0, 2:3, sl]
            dot = jax.lax.dot_general(
                qmat, ktb_ref[:, sl],
                (((1,), (0,)), ((), ())),
                preferred_element_type=jnp.float32)
            d2 = (a2 + b2_ref[0:1, sl]) - 2.0 * dot
            msk = d2 < best
            best = jnp.where(msk, d2, best)
            bx = jnp.where(msk, jnp.broadcast_to(kxE, (n, _KCHUNK)), bx)
            by = jnp.where(msk, jnp.broadcast_to(kyE, (n, _KCHUNK)), by)
            bz = jnp.where(msk, jnp.broadcast_to(kzE, (n, _KCHUNK)), bz)
            return best, bx, by, bz

        init = (jnp.full((n, _KCHUNK), jnp.inf, jnp.float32),
                jnp.zeros((n, _KCHUNK), jnp.float32),
                jnp.zeros((n, _KCHUNK), jnp.float32),
                jnp.zeros((n, _KCHUNK), jnp.float32))
        best, bx, by, bz = jax.lax.fori_loop(0, nchunks, chunk_step, init)

        bmin = jnp.min(best, axis=1, keepdims=True)
        eq = best == bmin
        li = jnp.min(jnp.where(eq, lane, _KCHUNK), axis=1, keepdims=True)
        pick = lane == li
        cx = jnp.sum(jnp.where(pick, bx, 0.0), axis=1, keepdims=True)
        cy = jnp.sum(jnp.where(pick, by, 0.0), axis=1, keepdims=True)
        cz = jnp.sum(jnp.where(pick, bz, 0.0), axis=1, keepdims=True)

        d = jnp.sqrt(jnp.maximum(bmin, 1e-12))
        dmax = jnp.max(d)
        alpha = _ALPHA * (2.0 - d / (dmax + 1e-6))
        qx = qx + alpha * (cx - qx)
        qy = qy + alpha * (cy - qy)
        qz = qz + alpha * (cz - qz)

    out_ref[0] = jnp.concatenate([qx, qy, qz], axis=1)


def _tc_call(pred, partial):
    b, n, _ = pred.shape
    m = partial.shape[1]
    part_t = jnp.swapaxes(partial, 1, 2)
    return pl.pallas_call(
        _tc_body,
        grid=(b,),
        in_specs=[
            pl.BlockSpec((1, n, 3), lambda i: (i, 0, 0)),
            pl.BlockSpec((1, 3, m), lambda i: (i, 0, 0)),
        ],
        out_specs=pl.BlockSpec((1, n, 3), lambda i: (i, 0, 0)),
        out_shape=jax.ShapeDtypeStruct((b, n, 3), jnp.float32),
        scratch_shapes=[
            pltpu.VMEM((3, m), jnp.bfloat16),
            pltpu.VMEM((1, m), jnp.float32),
        ],
    )(pred, part_t)


_SCB = 4


def _device_kernel(scb, pred, partial):
    b = pred.shape[0]
    pred2 = pred.reshape(b, _N * 3)
    part2 = partial.reshape(b, _M * 3)
    parts = []
    if scb:
        sc_out = _make_sc_kernel(scb)(pred2[:scb], part2[:scb])
        parts.append(sc_out.reshape(scb, _N, 3))
    if b - scb:
        parts.append(_tc_call(pred[scb:], partial[scb:]))
    return jnp.concatenate(parts, axis=0) if len(parts) > 1 else parts[0]


def kernel(pred, partial):
    return _device_kernel(_SCB, pred, partial)
